# baseline (speedup 1.0000x reference)
import jax
import jax.numpy as jnp
from jax.experimental import pallas as pl
from jax.experimental.pallas import tpu as pltpu

E = 8
T = 2048
D = 2048
H = 1408
TM = 256
NT = T // TM
C13 = 2 * H // NT
C2 = D // NT


def _fused_body(x_ref, w13_hbm, w2_hbm, o_ref,
                w13a, w13b, w2a, w2b, stage13, stage2, sem13, sem2):
    e = pl.program_id(0)
    t = pl.program_id(1)
    cur = jax.lax.rem(e, 2)

    def copies(src_e, c, slot):
        cp13 = pltpu.make_async_copy(
            w13_hbm.at[src_e, pl.ds(c * C13, C13), :],
            stage13.at[slot], sem13.at[slot])
        cp2 = pltpu.make_async_copy(
            w2_hbm.at[src_e, pl.ds(c * C2, C2), :],
            stage2.at[slot], sem2.at[slot])
        return cp13, cp2

    def start(src_e, c, slot):
        cp13, cp2 = copies(src_e, c, slot)
        cp13.start()
        cp2.start()

    @pl.when((e == 0) & (t == 0))
    def _():
        start(0, 0, 0)
        for c in range(NT):
            if c + 1 < NT:
                start(0, c + 1, (c + 1) % 2)
            cp13, cp2 = copies(0, c, c % 2)
            cp13.wait()
            cp2.wait()
            w13a[pl.ds(c * C13, C13), :] = stage13[c % 2].astype(jnp.bfloat16)
            w2a[pl.ds(c * C2, C2), :] = stage2[c % 2].astype(jnp.bfloat16)
        start(1, 0, 0)

    def step(ring13_rd, ring2_rd, ring13_wr, ring2_wr):
        nc = jax.lax.rem(t + 1, NT)
        ne = jnp.minimum(jnp.where(t < NT - 1, e + 1, e + 2), E - 1)

        @pl.when(~((e == E - 1) & (t == NT - 1)))
        def _():
            start(ne, nc, jax.lax.rem(t + 1, 2))

        cp13, cp2 = copies(jnp.minimum(e + 1, E - 1), t,
                           jax.lax.rem(t, 2))
        cp13.wait()
        cp2.wait()

        xb = x_ref[...].astype(jnp.bfloat16)
        dn = (((1,), (1,)), ((), ()))

        cuts = (0, 512, 896, H)

        def gemm1(lo, hi):
            g = jax.lax.dot_general(
                xb, ring13_rd[lo:hi, :], dn,
                preferred_element_type=jnp.float32)
            u = jax.lax.dot_general(
                xb, ring13_rd[H + lo:H + hi, :], dn,
                preferred_element_type=jnp.float32)
            return g, u

        def gemm2(hb, lo, hi):
            return jax.lax.dot_general(
                hb, ring2_rd[:, lo:hi], dn,
                preferred_element_type=jnp.float32)

        def swiglu(gu):
            g, u = gu
            return ((g * jax.nn.sigmoid(g)) * u).astype(jnp.bfloat16)

        gu_a = gemm1(cuts[0], cuts[1])
        hb_a = swiglu(gu_a)
        gu_b = gemm1(cuts[1], cuts[2])
        o_a = gemm2(hb_a, cuts[0], cuts[1])
        hb_b = swiglu(gu_b)
        gu_c = gemm1(cuts[2], cuts[3])
        o_b = gemm2(hb_b, cuts[1], cuts[2])
        hb_c = swiglu(gu_c)

        ring13_wr[pl.ds(t * C13, C13), :] = (
            stage13[jax.lax.rem(t, 2)].astype(jnp.bfloat16))
        ring2_wr[pl.ds(t * C2, C2), :] = (
            stage2[jax.lax.rem(t, 2)].astype(jnp.bfloat16))

        o_c = gemm2(hb_c, cuts[2], cuts[3])
        o_ref[...] = (o_a + o_b) + o_c

    @pl.when(cur == 0)
    def _():
        step(w13a, w2a, w13b, w2b)

    @pl.when(cur == 1)
    def _():
        step(w13b, w2b, w13a, w2a)


def kernel(x, w13, w2, num_tokens_per_expert):
    out = pl.pallas_call(
        _fused_body,
        grid=(E, NT),
        in_specs=[
            pl.BlockSpec((TM, D), lambda e, t: (e * NT + t, 0)),
            pl.BlockSpec(memory_space=pl.ANY),
            pl.BlockSpec(memory_space=pl.ANY),
        ],
        out_specs=pl.BlockSpec((TM, D), lambda e, t: (e * NT + t, 0)),
        out_shape=jax.ShapeDtypeStruct((E * T, D), jnp.float32),
        scratch_shapes=[
            pltpu.VMEM((2 * H, D), jnp.bfloat16),
            pltpu.VMEM((2 * H, D), jnp.bfloat16),
            pltpu.VMEM((D, H), jnp.bfloat16),
            pltpu.VMEM((D, H), jnp.bfloat16),
            pltpu.VMEM((2, C13, D), jnp.float32),
            pltpu.VMEM((2, C2, H), jnp.float32),
            pltpu.SemaphoreType.DMA((2,)),
            pltpu.SemaphoreType.DMA((2,)),
        ],
        compiler_params=pltpu.CompilerParams(
            dimension_semantics=("parallel", "arbitrary")),
    )(x, w13, w2)
    return out


# speedup vs baseline: 1.0142x; 1.0142x over previous
import jax
import jax.numpy as jnp
from jax.experimental import pallas as pl
from jax.experimental.pallas import tpu as pltpu

E = 8
T = 2048
D = 2048
H = 1408
TM = 256
NT = T // TM
C13 = 2 * H // NT
C2 = D // NT


def _fused_body(x_ref, w13_hbm, w2_hbm, o_ref,
                w13a, w13b, w2a, w2b, stage13, stage2, sem13, sem2):
    e = pl.program_id(0)
    t = pl.program_id(1)
    cur = jax.lax.rem(e, 2)

    def copies(src_e, c, slot):
        cp13 = pltpu.make_async_copy(
            w13_hbm.at[src_e, pl.ds(c * C13, C13), :],
            stage13.at[slot], sem13.at[slot])
        cp2 = pltpu.make_async_copy(
            w2_hbm.at[src_e, pl.ds(c * C2, C2), :],
            stage2.at[slot], sem2.at[slot])
        return cp13, cp2

    def start(src_e, c, slot):
        cp13, cp2 = copies(src_e, c, slot)
        cp13.start()
        cp2.start()

    @pl.when((e == 0) & (t == 0))
    def _():
        start(0, 0, 0)
        for c in range(NT):
            if c + 1 < NT:
                start(0, c + 1, (c + 1) % 2)
            cp13, cp2 = copies(0, c, c % 2)
            cp13.wait()
            cp2.wait()
            w13a[pl.ds(c * C13, C13), :] = stage13[c % 2].astype(jnp.bfloat16)
            w2a[pl.ds(c * C2, C2), :] = stage2[c % 2].astype(jnp.bfloat16)
        start(1, 0, 0)

    def step(ring13_rd, ring2_rd, ring13_wr, ring2_wr):
        nc = jax.lax.rem(t + 1, NT)
        ne = jnp.minimum(jnp.where(t < NT - 1, e + 1, e + 2), E - 1)

        @pl.when(~((e == E - 1) & (t == NT - 1)))
        def _():
            start(ne, nc, jax.lax.rem(t + 1, 2))

        cp13, cp2 = copies(jnp.minimum(e + 1, E - 1), t,
                           jax.lax.rem(t, 2))
        cp13.wait()
        cp2.wait()

        xb = x_ref[...].astype(jnp.bfloat16)
        dn = (((1,), (1,)), ((), ()))

        HA = 768
        g_a = jax.lax.dot_general(
            xb, ring13_rd[0:HA, :], dn,
            preferred_element_type=jnp.float32)
        u_a = jax.lax.dot_general(
            xb, ring13_rd[H:H + HA, :], dn,
            preferred_element_type=jnp.float32)
        hb_a = ((g_a * jax.nn.sigmoid(g_a)) * u_a).astype(jnp.bfloat16)

        g_b = jax.lax.dot_general(
            xb, ring13_rd[HA:H, :], dn,
            preferred_element_type=jnp.float32)
        u_b = jax.lax.dot_general(
            xb, ring13_rd[H + HA:2 * H, :], dn,
            preferred_element_type=jnp.float32)

        ring13_wr[pl.ds(t * C13, C13), :] = (
            stage13[jax.lax.rem(t, 2)].astype(jnp.bfloat16))
        ring2_wr[pl.ds(t * C2, C2), :] = (
            stage2[jax.lax.rem(t, 2)].astype(jnp.bfloat16))

        o_a = jax.lax.dot_general(
            hb_a, ring2_rd[:, 0:HA], dn,
            preferred_element_type=jnp.float32)
        hb_b = ((g_b * jax.nn.sigmoid(g_b)) * u_b).astype(jnp.bfloat16)
        o_b = jax.lax.dot_general(
            hb_b, ring2_rd[:, HA:H], dn,
            preferred_element_type=jnp.float32)
        o_ref[...] = o_a + o_b

    @pl.when(cur == 0)
    def _():
        step(w13a, w2a, w13b, w2b)

    @pl.when(cur == 1)
    def _():
        step(w13b, w2b, w13a, w2a)


def kernel(x, w13, w2, num_tokens_per_expert):
    out = pl.pallas_call(
        _fused_body,
        grid=(E, NT),
        in_specs=[
            pl.BlockSpec((TM, D), lambda e, t: (e * NT + t, 0)),
            pl.BlockSpec(memory_space=pl.ANY),
            pl.BlockSpec(memory_space=pl.ANY),
        ],
        out_specs=pl.BlockSpec((TM, D), lambda e, t: (e * NT + t, 0)),
        out_shape=jax.ShapeDtypeStruct((E * T, D), jnp.float32),
        scratch_shapes=[
            pltpu.VMEM((2 * H, D), jnp.bfloat16),
            pltpu.VMEM((2 * H, D), jnp.bfloat16),
            pltpu.VMEM((D, H), jnp.bfloat16),
            pltpu.VMEM((D, H), jnp.bfloat16),
            pltpu.VMEM((2, C13, D), jnp.float32),
            pltpu.VMEM((2, C2, H), jnp.float32),
            pltpu.SemaphoreType.DMA((2,)),
            pltpu.SemaphoreType.DMA((2,)),
        ],
        compiler_params=pltpu.CompilerParams(
            dimension_semantics=("parallel", "arbitrary")),
    )(x, w13, w2)
    return out
